# baseline (speedup 1.0000x reference)
import jax
import jax.numpy as jnp
from jax.experimental import pallas as pl
from jax.experimental.pallas import tpu as pltpu

B, S, D, E, DH, DR, TOPK = 2, 2048, 1024, 8, 512, 256, 2
F, DE = 16, 32
FD = F * DE
K = D + FD
T = B * S
TT = 512
NT = T // TT


def _moe_body(h_ref, f_ref, fw_ref, fb_ref, rw1_ref, rb1_ref, rw2_ref, rb2_ref,
              ew1_ref, eb1_ref, ew2_ref, eb2_ref,
              delta_ref, gates_ref, logits_ref):
    h = h_ref[:]
    ft = f_ref[:]
    femb = ft[:, :, None] * fw_ref[:][None, :, :] + fb_ref[:][None, :, :]
    fx = femb.reshape(TT, FD)

    pre = jnp.dot(h, rw1_ref[:D], preferred_element_type=jnp.float32)
    pre += jnp.dot(fx, rw1_ref[D:], preferred_element_type=jnp.float32)
    rh = jax.nn.gelu(pre + rb1_ref[:])
    logits = jnp.dot(rh, rw2_ref[:], preferred_element_type=jnp.float32) + rb2_ref[:]
    logits_ref[:] = logits

    m1 = jnp.max(logits, axis=-1, keepdims=True)
    is_max = logits == m1
    cnt = jnp.sum(is_max.astype(jnp.int32), axis=-1, keepdims=True)
    excl = jnp.max(jnp.where(is_max, -jnp.inf, logits), axis=-1, keepdims=True)
    thresh = jnp.where(cnt >= TOPK, m1, excl)
    masked = jnp.where(logits >= thresh, logits, -1e9)
    gates = jax.nn.softmax(masked, axis=-1)
    gates_ref[:] = gates

    acc = jnp.dot(gates, eb2_ref[:], preferred_element_type=jnp.float32)
    for e in range(E):
        fe = fx[:, DE * e: DE * e + 4 * DE]
        pre_e = jnp.dot(h, ew1_ref[e, :D, :], preferred_element_type=jnp.float32)
        pre_e += jnp.dot(fe, ew1_ref[e, D:, :], preferred_element_type=jnp.float32)
        he = jax.nn.gelu(pre_e + eb1_ref[e:e + 1, :])
        he = he * gates[:, e:e + 1]
        acc += jnp.dot(he, ew2_ref[e], preferred_element_type=jnp.float32)
    delta_ref[:] = acc


def kernel(hidden, feat, feat_W, feat_b, router_W1, router_b1, router_W2,
           router_b2, expert_W1, expert_b1, expert_W2, expert_b2):
    const = lambda *dims: pl.BlockSpec(dims, lambda t: (0,) * len(dims))
    delta2d, gates2d, logits2d = pl.pallas_call(
        _moe_body,
        grid=(NT,),
        in_specs=[
            pl.BlockSpec((TT, D), lambda t: (t, 0)),
            pl.BlockSpec((TT, F), lambda t: (t, 0)),
            const(F, DE),
            const(F, DE),
            const(K, DR),
            const(1, DR),
            const(DR, E),
            const(1, E),
            const(E, D + 4 * DE, DH),
            const(E, DH),
            const(E, DH, D),
            const(E, D),
        ],
        out_specs=[
            pl.BlockSpec((TT, D), lambda t: (t, 0)),
            pl.BlockSpec((TT, E), lambda t: (t, 0)),
            pl.BlockSpec((TT, E), lambda t: (t, 0)),
        ],
        out_shape=[
            jax.ShapeDtypeStruct((T, D), jnp.float32),
            jax.ShapeDtypeStruct((T, E), jnp.float32),
            jax.ShapeDtypeStruct((T, E), jnp.float32),
        ],
    )(hidden.reshape(T, D), feat.reshape(T, F), feat_W, feat_b,
      router_W1, router_b1.reshape(1, DR), router_W2, router_b2.reshape(1, E),
      expert_W1, expert_b1, expert_W2, expert_b2)

    return (delta2d.reshape(B, S, D),
            gates2d.reshape(B, S, E),
            logits2d.reshape(B, S, E))


# speedup vs baseline: 1.0588x; 1.0588x over previous
import jax
import jax.numpy as jnp
from jax.experimental import pallas as pl
from jax.experimental.pallas import tpu as pltpu

B, S, D, E, DH, DR, TOPK = 2, 2048, 1024, 8, 512, 256, 2
F, DE = 16, 32
FD = F * DE
K = D + FD
T = B * S
TT = 1024
NT = T // TT


def _moe_body(h_ref, f_ref, fw_ref, fb_ref, rw1_ref, rb1_ref, rw2_ref, rb2_ref,
              ew1_ref, eb1_ref, ew2_ref, eb2_ref,
              delta_ref, gates_ref, logits_ref):
    h = h_ref[:]
    ft = f_ref[:]
    femb = ft[:, :, None] * fw_ref[:][None, :, :] + fb_ref[:][None, :, :]
    fx = femb.reshape(TT, FD)

    pre = jnp.dot(h, rw1_ref[:D], preferred_element_type=jnp.float32)
    pre += jnp.dot(fx, rw1_ref[D:], preferred_element_type=jnp.float32)
    rh = jax.nn.gelu(pre + rb1_ref[:])
    logits = jnp.dot(rh, rw2_ref[:], preferred_element_type=jnp.float32) + rb2_ref[:]
    logits_ref[:] = logits

    m1 = jnp.max(logits, axis=-1, keepdims=True)
    is_max = logits == m1
    cnt = jnp.sum(is_max.astype(jnp.int32), axis=-1, keepdims=True)
    excl = jnp.max(jnp.where(is_max, -jnp.inf, logits), axis=-1, keepdims=True)
    thresh = jnp.where(cnt >= TOPK, m1, excl)
    masked = jnp.where(logits >= thresh, logits, -1e9)
    gates = jax.nn.softmax(masked, axis=-1)
    gates_ref[:] = gates

    hb = h.astype(jnp.bfloat16)
    fxb = fx.astype(jnp.bfloat16)
    acc = jnp.dot(gates, eb2_ref[:], preferred_element_type=jnp.float32)
    for e in range(E):
        fe = fxb[:, DE * e: DE * e + 4 * DE]
        pre_e = jnp.dot(hb, ew1_ref[e, :D, :], preferred_element_type=jnp.float32)
        pre_e += jnp.dot(fe, ew1_ref[e, D:, :], preferred_element_type=jnp.float32)
        he = jax.nn.gelu(pre_e + eb1_ref[e:e + 1, :])
        he = (he * gates[:, e:e + 1]).astype(jnp.bfloat16)
        acc += jnp.dot(he, ew2_ref[e], preferred_element_type=jnp.float32)
    delta_ref[:] = acc


def kernel(hidden, feat, feat_W, feat_b, router_W1, router_b1, router_W2,
           router_b2, expert_W1, expert_b1, expert_W2, expert_b2):
    const = lambda *dims: pl.BlockSpec(dims, lambda t: (0,) * len(dims))
    delta2d, gates2d, logits2d = pl.pallas_call(
        _moe_body,
        grid=(NT,),
        in_specs=[
            pl.BlockSpec((TT, D), lambda t: (t, 0)),
            pl.BlockSpec((TT, F), lambda t: (t, 0)),
            const(F, DE),
            const(F, DE),
            const(K, DR),
            const(1, DR),
            const(DR, E),
            const(1, E),
            const(E, D + 4 * DE, DH),
            const(E, DH),
            const(E, DH, D),
            const(E, D),
        ],
        out_specs=[
            pl.BlockSpec((TT, D), lambda t: (t, 0)),
            pl.BlockSpec((TT, E), lambda t: (t, 0)),
            pl.BlockSpec((TT, E), lambda t: (t, 0)),
        ],
        out_shape=[
            jax.ShapeDtypeStruct((T, D), jnp.float32),
            jax.ShapeDtypeStruct((T, E), jnp.float32),
            jax.ShapeDtypeStruct((T, E), jnp.float32),
        ],
    )(hidden.reshape(T, D), feat.reshape(T, F), feat_W, feat_b,
      router_W1, router_b1.reshape(1, DR), router_W2, router_b2.reshape(1, E),
      expert_W1.astype(jnp.bfloat16), expert_b1,
      expert_W2.astype(jnp.bfloat16), expert_b2)

    return (delta2d.reshape(B, S, D),
            gates2d.reshape(B, S, E),
            logits2d.reshape(B, S, E))
